# baseline (speedup 1.0000x reference)
import functools

import jax
import jax.numpy as jnp
from jax import lax
from jax.experimental import pallas as pl
from jax.experimental.pallas import tpu as pltpu
from jax.experimental.pallas import tpu_sc as plsc

_B = 256
_D = 128

_smesh = plsc.ScalarSubcoreMesh(axis_name="c", num_cores=1)


@functools.partial(
    pl.kernel,
    mesh=_smesh,
    out_type=jax.ShapeDtypeStruct((_B, _D), jnp.float32),
    scratch_types=[
        pltpu.SMEM((_B,), jnp.int32),
        pltpu.SemaphoreType.DMA,
    ],
)
def _gather(table_hbm, idx_hbm, out_hbm, idx_s, sem):
    pltpu.sync_copy(idx_hbm, idx_s)

    def body(i, _):
        row = idx_s[i]
        pltpu.make_async_copy(
            table_hbm.at[pl.ds(row, 1)], out_hbm.at[pl.ds(i, 1)], sem
        ).start()
        return 0

    lax.fori_loop(0, _B, body, 0)
    pltpu.make_async_copy(table_hbm.at[pl.ds(0, _B)], out_hbm, sem).wait()


def kernel(layer_input, ordinals):
    return _gather(layer_input, ordinals)
